# baseline (speedup 1.0000x reference)
import functools

import jax
import jax.numpy as jnp
from jax import lax
from jax.experimental import pallas as pl
from jax.experimental.pallas import tpu as pltpu
from jax.experimental.pallas import tpu_sc as plsc

B = 16384
D = 8
NROWS = 1_000_000
NC = 2
NS = 16
NW = NC * NS
BPW = B // NW
CH = 128
NCH = BPW // CH


@functools.lru_cache(maxsize=1)
def _make_sc_gather():
    mesh = plsc.VectorSubcoreMesh(core_axis_name="c", subcore_axis_name="s")

    @functools.partial(
        pl.kernel,
        mesh=mesh,
        compiler_params=pltpu.CompilerParams(use_tc_tiling_on_sc=False),
        out_type=[jax.ShapeDtypeStruct((D, B), jnp.float32) for _ in range(4)],
        scratch_types=[
            pltpu.VMEM((NCH, CH), jnp.int32),
            pltpu.VMEM((NCH, CH), jnp.int32),
            pltpu.VMEM((D, BPW), jnp.float32),
            pltpu.VMEM((D, BPW), jnp.float32),
            pltpu.VMEM((D, BPW), jnp.float32),
            pltpu.VMEM((D, BPW), jnp.float32),
            pltpu.SemaphoreType.DMA,
        ],
    )
    def _sc_gather(users_hbm, items_hbm, guT_hbm, giT_hbm, muT_hbm, miT_hbm,
                   o_gu, o_gi, o_mu, o_mi,
                   uidx, iidx, t_gu, t_gi, t_mu, t_mi, sem):
        wid = lax.axis_index("s") * NC + lax.axis_index("c")
        base = wid * BPW
        pltpu.sync_copy(users_hbm.at[pl.ds(wid * NCH, NCH)], uidx)
        pltpu.sync_copy(items_hbm.at[pl.ds(wid * NCH, NCH)], iidx)
        copies = []
        for k in range(NCH):
            for c in range(D):
                dsl = pl.ds(k * CH, CH)
                uo = uidx.at[k]
                io = iidx.at[k]
                copies.append(pltpu.async_copy(
                    guT_hbm.at[c].at[uo], t_gu.at[c, dsl], sem))
                copies.append(pltpu.async_copy(
                    giT_hbm.at[c].at[io], t_gi.at[c, dsl], sem))
                copies.append(pltpu.async_copy(
                    muT_hbm.at[c].at[uo], t_mu.at[c, dsl], sem))
                copies.append(pltpu.async_copy(
                    miT_hbm.at[c].at[io], t_mi.at[c, dsl], sem))
        for cp in copies:
            cp.wait()
        out_sl = pl.ds(base, BPW)
        pltpu.sync_copy(t_gu, o_gu.at[:, out_sl])
        pltpu.sync_copy(t_gi, o_gi.at[:, out_sl])
        pltpu.sync_copy(t_mu, o_mu.at[:, out_sl])
        pltpu.sync_copy(t_mi, o_mi.at[:, out_sl])

    return _sc_gather


TC_BLK = 2048


def _tc_body(guT_ref, giT_ref, muT_ref, miT_ref,
             W0uT_ref, W0iT_ref, b0_ref, W1T_ref, b1_ref, W2T_ref, b2_ref,
             W3T_ref, b3_ref, WohT_ref, WogT_ref, bo_ref, out_ref):
    muT = muT_ref[...]
    miT = miT_ref[...]
    h = W0uT_ref[...] @ muT + W0iT_ref[...] @ miT + b0_ref[...]
    h = jnp.maximum(h, 0.0)
    h = jnp.maximum(W1T_ref[...] @ h + b1_ref[...], 0.0)
    h = jnp.maximum(W2T_ref[...] @ h + b2_ref[...], 0.0)
    h = jnp.maximum(W3T_ref[...] @ h + b3_ref[...], 0.0)
    g = guT_ref[...] * giT_ref[...]
    logit = WohT_ref[...] @ h + WogT_ref[...] @ g + bo_ref[...]
    out_ref[...] = jax.nn.sigmoid(logit)


def kernel(users, items, gmf_user_table, gmf_item_table, mlp_user_table,
           mlp_item_table, W0, b0, W1, b1, W2, b2, W3, b3, W_out, b_out):
    users_r = users.astype(jnp.int32).reshape(B // CH, CH)
    items_r = items.astype(jnp.int32).reshape(B // CH, CH)
    guT, giT, muT, miT = _make_sc_gather()(
        users_r, items_r,
        gmf_user_table.T, gmf_item_table.T, mlp_user_table.T,
        mlp_item_table.T)

    grid = B // TC_BLK
    data_spec = pl.BlockSpec((D, TC_BLK), lambda i: (0, i))

    def wspec(shape):
        return pl.BlockSpec(shape, lambda i: tuple(0 for _ in shape))

    W0uT = W0[:D, :].T
    W0iT = W0[D:, :].T
    WohT = W_out[:D, :].T
    WogT = W_out[D:, :].T
    predT = pl.pallas_call(
        _tc_body,
        grid=(grid,),
        in_specs=[
            data_spec, data_spec, data_spec, data_spec,
            wspec(W0uT.shape), wspec(W0iT.shape), wspec((b0.shape[0], 1)),
            wspec(W1.T.shape), wspec((b1.shape[0], 1)),
            wspec(W2.T.shape), wspec((b2.shape[0], 1)),
            wspec(W3.T.shape), wspec((b3.shape[0], 1)),
            wspec(WohT.shape), wspec(WogT.shape), wspec((1, 1)),
        ],
        out_specs=pl.BlockSpec((1, TC_BLK), lambda i: (0, i)),
        out_shape=jax.ShapeDtypeStruct((1, B), jnp.float32),
    )(guT, giT, muT, miT,
      W0uT, W0iT, b0.reshape(-1, 1), W1.T, b1.reshape(-1, 1),
      W2.T, b2.reshape(-1, 1), W3.T, b3.reshape(-1, 1),
      WohT, WogT, b_out.reshape(1, 1))
    return predT.reshape(B, 1)
